# baseline (speedup 1.0000x reference)
import functools

import jax
import jax.numpy as jnp
from jax import lax
from jax.experimental import pallas as pl
from jax.experimental.pallas import tpu as pltpu
from jax.experimental.pallas import tpu_sc as plsc

B, C, T = 128, 2, 122880
T_OUT = 23552
TH, LH = 960, 128
OH = 184
ROW = OH * C * LH
BLK = C * LH
NC, NS = 2, 16
NW = NC * NS
BPW = B // NW

JOBS = tuple(
    [(128 * k, 128, 8, 4096 * k) for k in range(5)]
    + [(640 + 96 * k, 96, 4, (80 + 24 * k) * BLK) for k in range(3)]
)
NJ = len(JOBS)
S3_TH, S3_CNT, S3_OROW = 928, 32, 152

IN_TH = 128

_mesh = plsc.VectorSubcoreMesh(core_axis_name="c", subcore_axis_name="s")


@functools.partial(
    pl.kernel,
    out_type=jax.ShapeDtypeStruct((B, ROW), jnp.float32),
    mesh=_mesh,
    scratch_types=[
        pltpu.VMEM((2, IN_TH * BLK), jnp.float32),
        pltpu.VMEM((ROW,), jnp.float32),
        pltpu.SemaphoreType.DMA,
        pltpu.SemaphoreType.DMA,
        pltpu.SemaphoreType.DMA,
        pltpu.SemaphoreType.DMA,
    ],
    compiler_params=pltpu.CompilerParams(
        needs_layout_passes=False,
        use_tc_tiling_on_sc=False,
    ),
)
def _decimate(strain_hbm, out_hbm, in_v, row_v, si0, si1, so, s3):
    widx = lax.axis_index("s") * NC + lax.axis_index("c")
    iota = lax.iota(jnp.int32, 16)
    tl8 = iota * 8
    tl4a = iota * 4
    tl4b = iota * 4 + 64
    sin = (si0, si1)

    def start_in(b, j, buf):
        th0, cnt, _, _ = JOBS[j]
        return pltpu.async_copy(
            strain_hbm.at[b, pl.ds(th0 * BLK, cnt * BLK)],
            in_v.at[buf, pl.ds(0, cnt * BLK)],
            sin[buf],
        )

    def dec_chunk(j, buf):
        _, cnt, stride, ob0 = JOBS[j]
        src = in_v.at[buf]
        if stride == 8:
            @plsc.parallel_loop(0, cnt // 8, unroll=4)
            def dec8(i):
                sb = i * (8 * BLK)
                ob = ob0 + i * BLK
                for u in range(8):
                    for c in range(C):
                        g = plsc.load_gather(src, [tl8 + (sb + u * BLK + c * LH)])
                        row_v[pl.ds(ob + c * LH + u * 16, 16)] = g
        else:
            @plsc.parallel_loop(0, cnt // 4, unroll=4)
            def dec4(i):
                sb = i * (4 * BLK)
                ob = ob0 + i * BLK
                for u in range(4):
                    for c in range(C):
                        base = sb + u * BLK + c * LH
                        g = plsc.load_gather(src, [tl4a + base])
                        row_v[pl.ds(ob + c * LH + u * 32, 16)] = g
                        g = plsc.load_gather(src, [tl4b + base])
                        row_v[pl.ds(ob + c * LH + u * 32 + 16, 16)] = g

    out_cp = None
    for b_local in range(BPW):
        b = widx * BPW + b_local

        cps = [None] * NJ
        cps[0] = start_in(b, 0, 0)
        cps[1] = start_in(b, 1, 1)

        if out_cp is not None:
            out_cp.wait()

        c3 = pltpu.async_copy(
            strain_hbm.at[b, pl.ds(S3_TH * BLK, S3_CNT * BLK)],
            row_v.at[pl.ds(S3_OROW * BLK, S3_CNT * BLK)],
            s3,
        )

        for j in range(NJ):
            buf = j & 1
            cps[j].wait()
            dec_chunk(j, buf)
            if j + 2 < NJ:
                cps[j + 2] = start_in(b, j + 2, buf)

        c3.wait()
        out_cp = pltpu.async_copy(row_v, out_hbm.at[b], so)

    out_cp.wait()


def kernel(strain):
    flat = strain.reshape(B, C, TH, LH).transpose(0, 2, 1, 3).reshape(B, TH * BLK)
    y = _decimate(flat)
    return y.reshape(B, OH, C, LH).transpose(0, 2, 1, 3).reshape(B, C, T_OUT)


# speedup vs baseline: 1.0120x; 1.0120x over previous
import functools

import jax
import jax.numpy as jnp
from jax import lax
from jax.experimental import pallas as pl
from jax.experimental.pallas import tpu as pltpu
from jax.experimental.pallas import tpu_sc as plsc

B, C, T = 128, 2, 122880
T_OUT = 23552
TH, LH = 960, 128
OH = 184
ROW = OH * C * LH
BLK = C * LH
NC, NS = 2, 16
NW = NC * NS
BPW = B // NW

JOBS = tuple(
    [(128 * k, 128, 8, 4096 * k) for k in range(5)]
    + [(640 + 96 * k, 96, 4, (80 + 24 * k) * BLK) for k in range(3)]
)
NJ = len(JOBS)
S3_TH, S3_CNT, S3_OROW = 928, 32, 152

IN_TH = 128

_mesh = plsc.VectorSubcoreMesh(core_axis_name="c", subcore_axis_name="s")


@functools.partial(
    pl.kernel,
    out_type=jax.ShapeDtypeStruct((B, ROW), jnp.float32),
    mesh=_mesh,
    scratch_types=[
        pltpu.VMEM((2, IN_TH * BLK), jnp.float32),
        pltpu.VMEM((ROW,), jnp.float32),
        pltpu.SemaphoreType.DMA,
        pltpu.SemaphoreType.DMA,
        pltpu.SemaphoreType.DMA,
        pltpu.SemaphoreType.DMA,
    ],
    compiler_params=pltpu.CompilerParams(
        needs_layout_passes=False,
        use_tc_tiling_on_sc=False,
    ),
)
def _decimate(strain_hbm, out_hbm, in_v, row_v, si0, si1, so, s3):
    widx = lax.axis_index("s") * NC + lax.axis_index("c")
    iota = lax.iota(jnp.int32, 16)
    tl8 = iota * 8
    tl4a = iota * 4
    tl4b = iota * 4 + 64
    sin = (si0, si1)

    def start_in(b, j, buf):
        th0, cnt, _, _ = JOBS[j]
        return pltpu.async_copy(
            strain_hbm.at[b, pl.ds(th0 * BLK, cnt * BLK)],
            in_v.at[buf, pl.ds(0, cnt * BLK)],
            sin[buf],
        )

    def dec_chunk(j, buf):
        _, cnt, stride, ob0 = JOBS[j]
        src = in_v.at[buf]
        if stride == 8:
            @plsc.parallel_loop(0, cnt // 8)
            def dec8(i):
                sb = i * (8 * BLK)
                ob = ob0 + i * BLK
                for u in range(8):
                    for c in range(C):
                        g = plsc.load_gather(src, [tl8 + (sb + u * BLK + c * LH)])
                        row_v[pl.ds(ob + c * LH + u * 16, 16)] = g
        else:
            @plsc.parallel_loop(0, cnt // 4)
            def dec4(i):
                sb = i * (4 * BLK)
                ob = ob0 + i * BLK
                for u in range(4):
                    for c in range(C):
                        base = sb + u * BLK + c * LH
                        g = plsc.load_gather(src, [tl4a + base])
                        row_v[pl.ds(ob + c * LH + u * 32, 16)] = g
                        g = plsc.load_gather(src, [tl4b + base])
                        row_v[pl.ds(ob + c * LH + u * 32 + 16, 16)] = g

    out_cp = None
    for b_local in range(BPW):
        b = widx * BPW + b_local

        cps = [None] * NJ
        cps[0] = start_in(b, 0, 0)
        cps[1] = start_in(b, 1, 1)

        if out_cp is not None:
            out_cp.wait()

        c3 = pltpu.async_copy(
            strain_hbm.at[b, pl.ds(S3_TH * BLK, S3_CNT * BLK)],
            row_v.at[pl.ds(S3_OROW * BLK, S3_CNT * BLK)],
            s3,
        )

        for j in range(NJ):
            buf = j & 1
            cps[j].wait()
            dec_chunk(j, buf)
            if j + 2 < NJ:
                cps[j + 2] = start_in(b, j + 2, buf)

        c3.wait()
        out_cp = pltpu.async_copy(row_v, out_hbm.at[b], so)

    out_cp.wait()


def kernel(strain):
    flat = strain.reshape(B, C, TH, LH).transpose(0, 2, 1, 3).reshape(B, TH * BLK)
    y = _decimate(flat)
    return y.reshape(B, OH, C, LH).transpose(0, 2, 1, 3).reshape(B, C, T_OUT)


# speedup vs baseline: 1.0289x; 1.0167x over previous
import functools

import jax
import jax.numpy as jnp
from jax import lax
from jax.experimental import pallas as pl
from jax.experimental.pallas import tpu as pltpu
from jax.experimental.pallas import tpu_sc as plsc

B, C, T = 128, 2, 122880
T_OUT = 23552
TH, LH = 960, 128
OH = 184
ROW = OH * C * LH
BLK = C * LH
NC, NS = 2, 16
NW = NC * NS
BPW = B // NW

JOBS = tuple(
    [(128 * k, 128, 8, 4096 * k) for k in range(5)]
    + [(640 + 96 * k, 96, 4, (80 + 24 * k) * BLK) for k in range(3)]
)
NJ = len(JOBS)
S3_TH, S3_CNT, S3_OROW = 928, 32, 152

IN_TH = 128

_mesh = plsc.VectorSubcoreMesh(core_axis_name="c", subcore_axis_name="s")


@functools.partial(
    pl.kernel,
    out_type=jax.ShapeDtypeStruct((B, ROW), jnp.float32),
    mesh=_mesh,
    scratch_types=[
        pltpu.VMEM((2, IN_TH * BLK), jnp.float32),
        pltpu.VMEM((ROW,), jnp.float32),
        pltpu.SemaphoreType.DMA,
        pltpu.SemaphoreType.DMA,
        pltpu.SemaphoreType.DMA,
        pltpu.SemaphoreType.DMA,
        pltpu.SemaphoreType.DMA,
    ],
    compiler_params=pltpu.CompilerParams(
        needs_layout_passes=False,
        use_tc_tiling_on_sc=False,
    ),
)
def _decimate(strain_hbm, out_hbm, in_v, row_v, si0, si1, so, sob, s3):
    widx = lax.axis_index("s") * NC + lax.axis_index("c")
    iota = lax.iota(jnp.int32, 16)
    tl8 = iota * 8
    tl4a = iota * 4
    tl4b = iota * 4 + 64
    sin = (si0, si1)

    def start_in(b, j, buf):
        th0, cnt, _, _ = JOBS[j]
        return pltpu.async_copy(
            strain_hbm.at[b, pl.ds(th0 * BLK, cnt * BLK)],
            in_v.at[buf, pl.ds(0, cnt * BLK)],
            sin[buf],
        )

    def dec_chunk(j, buf):
        _, cnt, stride, ob0 = JOBS[j]
        src = in_v.at[buf]
        if stride == 8:
            @plsc.parallel_loop(0, cnt // 8)
            def dec8(i):
                sb = i * (8 * BLK)
                ob = ob0 + i * BLK
                for u in range(8):
                    for c in range(C):
                        g = plsc.load_gather(src, [tl8 + (sb + u * BLK + c * LH)])
                        row_v[pl.ds(ob + c * LH + u * 16, 16)] = g
        else:
            @plsc.parallel_loop(0, cnt // 4)
            def dec4(i):
                sb = i * (4 * BLK)
                ob = ob0 + i * BLK
                for u in range(4):
                    for c in range(C):
                        base = sb + u * BLK + c * LH
                        g = plsc.load_gather(src, [tl4a + base])
                        row_v[pl.ds(ob + c * LH + u * 32, 16)] = g
                        g = plsc.load_gather(src, [tl4b + base])
                        row_v[pl.ds(ob + c * LH + u * 32 + 16, 16)] = g

    HALF = 80 * BLK
    out_a = None
    out_b = None
    for b_local in range(BPW):
        b = widx * BPW + b_local

        cps = [None] * NJ
        cps[0] = start_in(b, 0, 0)
        cps[1] = start_in(b, 1, 1)

        if out_b is not None:
            out_b.wait()

        c3 = pltpu.async_copy(
            strain_hbm.at[b, pl.ds(S3_TH * BLK, S3_CNT * BLK)],
            row_v.at[pl.ds(S3_OROW * BLK, S3_CNT * BLK)],
            s3,
        )

        for j in range(NJ):
            buf = j & 1
            cps[j].wait()
            if j == 0 and out_a is not None:
                out_a.wait()
            dec_chunk(j, buf)
            if j + 2 < NJ:
                cps[j + 2] = start_in(b, j + 2, buf)
            if j == 4:
                out_a = pltpu.async_copy(
                    row_v.at[pl.ds(0, HALF)], out_hbm.at[b, pl.ds(0, HALF)], so
                )

        c3.wait()
        out_b = pltpu.async_copy(
            row_v.at[pl.ds(HALF, ROW - HALF)],
            out_hbm.at[b, pl.ds(HALF, ROW - HALF)],
            sob,
        )

    out_a.wait()
    out_b.wait()


def kernel(strain):
    flat = strain.reshape(B, C, TH, LH).transpose(0, 2, 1, 3).reshape(B, TH * BLK)
    y = _decimate(flat)
    return y.reshape(B, OH, C, LH).transpose(0, 2, 1, 3).reshape(B, C, T_OUT)
